# baseline (speedup 1.0000x reference)
import jax
import jax.numpy as jnp
from jax import lax
from jax.experimental import pallas as pl
from jax.experimental.pallas import tpu as pltpu

_LOG2E = 1.4426950408889634
_LN2 = 0.6931471805599453


def _round_up(v, m):
    return (v + m - 1) // m * m


def _pad2d(a, rows, cols):
    r, c = a.shape
    if r == rows and c == cols:
        return a
    return jnp.pad(a, ((0, rows - r), (0, cols - c)))


def _fused_kernel(x_ref, mu_ref, rho_ref, eps_ref, b_ref, o_ref):
    t = jnp.exp2(rho_ref[...] * _LOG2E)
    sigma = jnp.log2(1.0 + t) * _LN2 + 1e-5
    w = (mu_ref[...] + eps_ref[...] * sigma).astype(jnp.bfloat16)
    xb = x_ref[...].astype(jnp.bfloat16)
    acc = lax.dot_general(
        xb, w,
        dimension_numbers=(((1,), (1,)), ((), ())),
        preferred_element_type=jnp.float32)
    o_ref[...] = acc + b_ref[...]


def _forward(x, mu, rho, eps, bias2d, Bp, Np, Kp, tm):
    return pl.pallas_call(
        _fused_kernel,
        out_shape=jax.ShapeDtypeStruct((Bp, Np), jnp.float32),
        grid=(Bp // tm,),
        in_specs=[
            pl.BlockSpec((tm, Kp), lambda i: (i, 0)),
            pl.BlockSpec((Np, Kp), lambda i: (0, 0)),
            pl.BlockSpec((Np, Kp), lambda i: (0, 0)),
            pl.BlockSpec((Np, Kp), lambda i: (0, 0)),
            pl.BlockSpec((1, Np), lambda i: (0, 0)),
        ],
        out_specs=pl.BlockSpec((tm, Np), lambda i: (i, 0)),
        compiler_params=pltpu.CompilerParams(
            dimension_semantics=("parallel",),
            vmem_limit_bytes=100 * 2**20),
    )(x, mu, rho, eps, bias2d)


@jax.jit
def kernel(x, mu, rho, eps, bias):
    B, in_f = x.shape
    out_f, _ = mu.shape

    x = x.astype(jnp.float32)
    mu = mu.astype(jnp.float32)
    rho = rho.astype(jnp.float32)
    eps = eps.astype(jnp.float32)
    bias = bias.astype(jnp.float32)

    Bp = _round_up(B, 256)
    Np = _round_up(out_f, 256)
    Kp = _round_up(in_f, 256)

    xp = _pad2d(x, Bp, Kp)
    mup = _pad2d(mu, Np, Kp)
    rhop = _pad2d(rho, Np, Kp)
    epsp = _pad2d(eps, Np, Kp)
    biasp = _pad2d(bias.reshape(1, out_f), 1, Np)

    tm = 512 if Bp % 512 == 0 else Bp
    out = _forward(xp, mup, rhop, epsp, biasp, Bp, Np, Kp, tm)

    if Bp != B or Np != out_f:
        out = out[:B, :out_f]
    return out


# speedup vs baseline: 1.1426x; 1.1426x over previous
import jax
import jax.numpy as jnp
from jax import lax
from jax.experimental import pallas as pl
from jax.experimental.pallas import tpu as pltpu

_LOG2E = 1.4426950408889634
_LN2 = 0.6931471805599453


def _round_up(v, m):
    return (v + m - 1) // m * m


def _pad2d(a, rows, cols):
    r, c = a.shape
    if r == rows and c == cols:
        return a
    return jnp.pad(a, ((0, rows - r), (0, cols - c)))


def _fused_kernel(x_ref, mu_ref, rho_ref, eps_ref, b_ref, o_ref, w_ref):
    @pl.when(pl.program_id(0) == 0)
    def _():
        t = jnp.exp2(rho_ref[...] * _LOG2E)
        sigma = jnp.log2(1.0 + t) * _LN2 + 1e-5
        w_ref[...] = (mu_ref[...] + eps_ref[...] * sigma).astype(jnp.bfloat16)

    xb = x_ref[...].astype(jnp.bfloat16)
    acc = lax.dot_general(
        xb, w_ref[...],
        dimension_numbers=(((1,), (1,)), ((), ())),
        preferred_element_type=jnp.float32)
    o_ref[...] = acc + b_ref[...]


def _forward(x, mu, rho, eps, bias2d, Bp, Np, Kp, tm):
    return pl.pallas_call(
        _fused_kernel,
        out_shape=jax.ShapeDtypeStruct((Bp, Np), jnp.float32),
        grid=(Bp // tm,),
        in_specs=[
            pl.BlockSpec((tm, Kp), lambda i: (i, 0)),
            pl.BlockSpec((Np, Kp), lambda i: (0, 0)),
            pl.BlockSpec((Np, Kp), lambda i: (0, 0)),
            pl.BlockSpec((Np, Kp), lambda i: (0, 0)),
            pl.BlockSpec((1, Np), lambda i: (0, 0)),
        ],
        out_specs=pl.BlockSpec((tm, Np), lambda i: (i, 0)),
        scratch_shapes=[pltpu.VMEM((Np, Kp), jnp.bfloat16)],
        compiler_params=pltpu.CompilerParams(
            dimension_semantics=("parallel",),
            vmem_limit_bytes=100 * 2**20),
    )(x, mu, rho, eps, bias2d)


@jax.jit
def kernel(x, mu, rho, eps, bias):
    B, in_f = x.shape
    out_f, _ = mu.shape

    x = x.astype(jnp.float32)
    mu = mu.astype(jnp.float32)
    rho = rho.astype(jnp.float32)
    eps = eps.astype(jnp.float32)
    bias = bias.astype(jnp.float32)

    Bp = _round_up(B, 256)
    Np = _round_up(out_f, 256)
    Kp = _round_up(in_f, 256)

    xp = _pad2d(x, Bp, Kp)
    mup = _pad2d(mu, Np, Kp)
    rhop = _pad2d(rho, Np, Kp)
    epsp = _pad2d(eps, Np, Kp)
    biasp = _pad2d(bias.reshape(1, out_f), 1, Np)

    tm = 512 if Bp % 512 == 0 else Bp
    out = _forward(xp, mup, rhop, epsp, biasp, Bp, Np, Kp, tm)

    if Bp != B or Np != out_f:
        out = out[:B, :out_f]
    return out


# speedup vs baseline: 1.2574x; 1.1004x over previous
import jax
import jax.numpy as jnp
from jax import lax
from jax.experimental import pallas as pl
from jax.experimental.pallas import tpu as pltpu

_LOG2E = 1.4426950408889634
_LN2 = 0.6931471805599453


def _round_up(v, m):
    return (v + m - 1) // m * m


def _pad2d(a, rows, cols):
    r, c = a.shape
    if r == rows and c == cols:
        return a
    return jnp.pad(a, ((0, rows - r), (0, cols - c)))


def _fused_kernel(x_ref, mu_ref, rho_ref, eps_ref, b_ref, o_ref, w_ref):
    @pl.when(pl.program_id(0) == 0)
    def _():
        t = jnp.exp2(rho_ref[...] * _LOG2E)
        sigma = jnp.log2(1.0 + t) * _LN2 + 1e-5
        w_ref[...] = (mu_ref[...] + eps_ref[...] * sigma).astype(jnp.bfloat16)

    xb = x_ref[...].astype(jnp.bfloat16)
    acc = lax.dot_general(
        xb, w_ref[...],
        dimension_numbers=(((1,), (1,)), ((), ())),
        preferred_element_type=jnp.float32)
    o_ref[...] = acc + b_ref[...]


def _forward(x, mu, rho, eps, bias2d, Bp, Np, Kp, tm):
    return pl.pallas_call(
        _fused_kernel,
        out_shape=jax.ShapeDtypeStruct((Bp, Np), jnp.float32),
        grid=(Bp // tm,),
        in_specs=[
            pl.BlockSpec((tm, Kp), lambda i: (i, 0)),
            pl.BlockSpec((Np, Kp), lambda i: (0, 0)),
            pl.BlockSpec((Np, Kp), lambda i: (0, 0)),
            pl.BlockSpec((Np, Kp), lambda i: (0, 0)),
            pl.BlockSpec((1, Np), lambda i: (0, 0)),
        ],
        out_specs=pl.BlockSpec((tm, Np), lambda i: (i, 0)),
        scratch_shapes=[pltpu.VMEM((Np, Kp), jnp.bfloat16)],
        compiler_params=pltpu.CompilerParams(
            dimension_semantics=("parallel",),
            vmem_limit_bytes=100 * 2**20),
    )(x, mu, rho, eps, bias2d)


@jax.jit
def kernel(x, mu, rho, eps, bias):
    B, in_f = x.shape
    out_f, _ = mu.shape

    x = x.astype(jnp.float32)
    mu = mu.astype(jnp.float32)
    rho = rho.astype(jnp.float32)
    eps = eps.astype(jnp.float32)
    bias = bias.astype(jnp.float32)

    Bp = _round_up(B, 256)
    Np = _round_up(out_f, 256)
    Kp = _round_up(in_f, 256)

    xp = _pad2d(x, Bp, Kp)
    mup = _pad2d(mu, Np, Kp)
    rhop = _pad2d(rho, Np, Kp)
    epsp = _pad2d(eps, Np, Kp)
    biasp = _pad2d(bias.reshape(1, out_f), 1, Np)

    tm = 1024 if Bp % 1024 == 0 else (512 if Bp % 512 == 0 else Bp)
    out = _forward(xp, mup, rhop, epsp, biasp, Bp, Np, Kp, tm)

    if Bp != B or Np != out_f:
        out = out[:B, :out_f]
    return out


# speedup vs baseline: 1.2911x; 1.0268x over previous
import jax
import jax.numpy as jnp
from jax import lax
from jax.experimental import pallas as pl
from jax.experimental.pallas import tpu as pltpu

_LOG2E = 1.4426950408889634
_LN2 = 0.6931471805599453
_NCH = 4


def _round_up(v, m):
    return (v + m - 1) // m * m


def _pad2d(a, rows, cols):
    r, c = a.shape
    if r == rows and c == cols:
        return a
    return jnp.pad(a, ((0, rows - r), (0, cols - c)))


def _manual_kernel(x_hbm, mu_hbm, rho_hbm, eps_hbm, b_hbm, o_hbm,
                   muv, rhov, epsv, bv, wv, xv, ov,
                   sem_w, sem_b, sem_x, sem_o):
    tm = xv.shape[1]
    cmu = pltpu.make_async_copy(mu_hbm, muv, sem_w.at[0])
    crho = pltpu.make_async_copy(rho_hbm, rhov, sem_w.at[1])
    ceps = pltpu.make_async_copy(eps_hbm, epsv, sem_w.at[2])
    cb = pltpu.make_async_copy(b_hbm, bv, sem_b)
    cmu.start(); crho.start(); ceps.start(); cb.start()
    cmu.wait(); crho.wait(); ceps.wait(); cb.wait()

    xcopies = []
    for i in range(_NCH):
        c = pltpu.make_async_copy(
            x_hbm.at[pl.ds(i * tm, tm), :], xv.at[i], sem_x.at[i])
        c.start()
        xcopies.append(c)

    t = jnp.exp2(rhov[...] * _LOG2E)
    sigma = jnp.log2(1.0 + t) * _LN2 + 1e-5
    wv[...] = (muv[...] + epsv[...] * sigma).astype(jnp.bfloat16)

    ocopies = []
    for i in range(_NCH):
        xcopies[i].wait()
        xb = xv[i].astype(jnp.bfloat16)
        acc = lax.dot_general(
            xb, wv[...],
            dimension_numbers=(((1,), (1,)), ((), ())),
            preferred_element_type=jnp.float32)
        ov[i] = acc + bv[...]
        c = pltpu.make_async_copy(
            ov.at[i], o_hbm.at[pl.ds(i * tm, tm), :], sem_o.at[i])
        c.start()
        ocopies.append(c)
    for c in ocopies:
        c.wait()


def _forward(x, mu, rho, eps, bias2d, Bp, Np, Kp):
    tm = Bp // _NCH
    return pl.pallas_call(
        _manual_kernel,
        out_shape=jax.ShapeDtypeStruct((Bp, Np), jnp.float32),
        in_specs=[pl.BlockSpec(memory_space=pl.ANY)] * 5,
        out_specs=pl.BlockSpec(memory_space=pl.ANY),
        scratch_shapes=[
            pltpu.VMEM((Np, Kp), jnp.float32),
            pltpu.VMEM((Np, Kp), jnp.float32),
            pltpu.VMEM((Np, Kp), jnp.float32),
            pltpu.VMEM((1, Np), jnp.float32),
            pltpu.VMEM((Np, Kp), jnp.bfloat16),
            pltpu.VMEM((_NCH, tm, Kp), jnp.float32),
            pltpu.VMEM((_NCH, tm, Np), jnp.float32),
            pltpu.SemaphoreType.DMA((3,)),
            pltpu.SemaphoreType.DMA,
            pltpu.SemaphoreType.DMA((_NCH,)),
            pltpu.SemaphoreType.DMA((_NCH,)),
        ],
        compiler_params=pltpu.CompilerParams(
            vmem_limit_bytes=100 * 2**20),
    )(x, mu, rho, eps, bias2d)


@jax.jit
def kernel(x, mu, rho, eps, bias):
    B, in_f = x.shape
    out_f, _ = mu.shape

    x = x.astype(jnp.float32)
    mu = mu.astype(jnp.float32)
    rho = rho.astype(jnp.float32)
    eps = eps.astype(jnp.float32)
    bias = bias.astype(jnp.float32)

    Bp = _round_up(B, 256 * _NCH)
    Np = _round_up(out_f, 256)
    Kp = _round_up(in_f, 256)

    xp = _pad2d(x, Bp, Kp)
    mup = _pad2d(mu, Np, Kp)
    rhop = _pad2d(rho, Np, Kp)
    epsp = _pad2d(eps, Np, Kp)
    biasp = _pad2d(bias.reshape(1, out_f), 1, Np)

    out = _forward(xp, mup, rhop, epsp, biasp, Bp, Np, Kp)

    if Bp != B or Np != out_f:
        out = out[:B, :out_f]
    return out


# speedup vs baseline: 1.3051x; 1.0109x over previous
import jax
import jax.numpy as jnp
from jax import lax
from jax.experimental import pallas as pl
from jax.experimental.pallas import tpu as pltpu

_LOG2E = 1.4426950408889634
_LN2 = 0.6931471805599453


def _round_up(v, m):
    return (v + m - 1) // m * m


def _pad2d(a, rows, cols):
    r, c = a.shape
    if r == rows and c == cols:
        return a
    return jnp.pad(a, ((0, rows - r), (0, cols - c)))


def _fused_kernel(x_ref, mu_ref, rho_ref, eps_ref, b_ref, o_ref):
    t = jnp.exp2(rho_ref[...] * _LOG2E)
    sigma = jnp.log2(1.0 + t) * _LN2 + 1e-5
    w = (mu_ref[...] + eps_ref[...] * sigma).astype(jnp.bfloat16)
    xb = x_ref[...].astype(jnp.bfloat16)
    acc = lax.dot_general(
        xb, w,
        dimension_numbers=(((1,), (1,)), ((), ())),
        preferred_element_type=jnp.float32)
    o_ref[...] = acc + b_ref[...]


def _forward(x, mu, rho, eps, bias2d, Bp, Np, Kp, tm):
    return pl.pallas_call(
        _fused_kernel,
        out_shape=jax.ShapeDtypeStruct((Bp, Np), jnp.float32),
        grid=(Bp // tm,),
        in_specs=[
            pl.BlockSpec((tm, Kp), lambda i: (i, 0)),
            pl.BlockSpec((Np, Kp), lambda i: (0, 0)),
            pl.BlockSpec((Np, Kp), lambda i: (0, 0)),
            pl.BlockSpec((Np, Kp), lambda i: (0, 0)),
            pl.BlockSpec((1, Np), lambda i: (0, 0)),
        ],
        out_specs=pl.BlockSpec((tm, Np), lambda i: (i, 0)),
        compiler_params=pltpu.CompilerParams(
            dimension_semantics=("parallel",),
            vmem_limit_bytes=100 * 2**20),
    )(x, mu, rho, eps, bias2d)


@jax.jit
def kernel(x, mu, rho, eps, bias):
    B, in_f = x.shape
    out_f, _ = mu.shape

    x = x.astype(jnp.float32)
    mu = mu.astype(jnp.float32)
    rho = rho.astype(jnp.float32)
    eps = eps.astype(jnp.float32)
    bias = bias.astype(jnp.float32)

    Bp = _round_up(B, 256)
    Np = _round_up(out_f, 256)
    Kp = _round_up(in_f, 256)

    xp = _pad2d(x, Bp, Kp)
    mup = _pad2d(mu, Np, Kp)
    rhop = _pad2d(rho, Np, Kp)
    epsp = _pad2d(eps, Np, Kp)
    biasp = _pad2d(bias.reshape(1, out_f), 1, Np)

    tm = 1024 if Bp % 1024 == 0 else (512 if Bp % 512 == 0 else Bp)
    out = _forward(xp, mup, rhop, epsp, biasp, Bp, Np, Kp, tm)

    if Bp != B or Np != out_f:
        out = out[:B, :out_f]
    return out
